# baseline (speedup 1.0000x reference)
import functools

import jax
import jax.numpy as jnp
from jax.experimental import pallas as pl
from jax.experimental.pallas import tpu as pltpu
from jax.experimental.pallas import tpu_sc as plsc


def kernel(x, pos_embedding):
    B, S, D = x.shape
    W = 4
    mesh = plsc.VectorSubcoreMesh(core_axis_name="core", subcore_axis_name="subcore")

    @functools.partial(
        pl.kernel,
        out_type=jax.ShapeDtypeStruct((B, S, D), x.dtype),
        mesh=mesh,
    )
    def sc_k(x_hbm, e_hbm, o_hbm):
        def body(x_v, e_v, o_v):
            @pl.loop(0, B)
            def _(b):
                @pl.loop(0, W)
                def _(r):
                    for c in range(0, D, 16):
                        o_v.at[b, r, pl.ds(c, 16)][...] = (
                            x_v.at[b, r, pl.ds(c, 16)][...]
                            + e_v.at[r, pl.ds(c, 16)][...]
                        )

        pltpu.emit_pipeline(
            body,
            grid=(S // W,),
            in_specs=[
                pl.BlockSpec((B, W, D), lambda i: (0, i, 0)),
                pl.BlockSpec((W, D), lambda i: (i, 0)),
            ],
            out_specs=[pl.BlockSpec((B, W, D), lambda i: (0, i, 0))],
            core_axis_name=("core", "subcore"),
            dimension_semantics=(pltpu.PARALLEL,),
        )(x_hbm, e_hbm, o_hbm)

    return sc_k(x, pos_embedding)


# speedup vs baseline: 1.0034x; 1.0034x over previous
import functools

import jax
import jax.numpy as jnp
from jax.experimental import pallas as pl
from jax.experimental.pallas import tpu as pltpu
from jax.experimental.pallas import tpu_sc as plsc


def kernel(x, pos_embedding):
    B, S, D = x.shape
    W = 16
    mesh = plsc.VectorSubcoreMesh(core_axis_name="core", subcore_axis_name="subcore")

    @functools.partial(
        pl.kernel,
        out_type=jax.ShapeDtypeStruct((B, S, D), x.dtype),
        mesh=mesh,
    )
    def sc_k(x_hbm, e_hbm, o_hbm):
        def body(x_v, e_v, o_v):
            @pl.loop(0, W)
            def _(r):
                for c in range(0, D, 16):
                    o_v.at[0, r, pl.ds(c, 16)][...] = (
                        x_v.at[0, r, pl.ds(c, 16)][...]
                        + e_v.at[r, pl.ds(c, 16)][...]
                    )

        pltpu.emit_pipeline(
            body,
            grid=(S // W, B),
            in_specs=[
                pl.BlockSpec((1, W, D), lambda i, b: (b, i, 0)),
                pl.BlockSpec((W, D), lambda i, b: (i, 0)),
            ],
            out_specs=[pl.BlockSpec((1, W, D), lambda i, b: (b, i, 0))],
            core_axis_name=("core", "subcore"),
            dimension_semantics=(pltpu.PARALLEL, pltpu.ARBITRARY),
        )(x_hbm, e_hbm, o_hbm)

    return sc_k(x, pos_embedding)
